# baseline (speedup 1.0000x reference)
import functools

import jax
import jax.numpy as jnp
from jax import lax
from jax.experimental import pallas as pl
from jax.experimental.pallas import tpu as pltpu
from jax.experimental.pallas import tpu_sc as plsc


@functools.lru_cache(maxsize=None)
def _make_gather(V, D, B):
  info = plsc.get_sparse_core_info()
  NC, NS = info.num_cores, info.num_subcores
  NW = NC * NS
  assert B % NW == 0
  b_per_w = B // NW
  K = 16
  NBUF = 3
  n_chunks = b_per_w // K
  assert b_per_w % K == 0 and n_chunks >= 2 * NBUF
  mesh = plsc.VectorSubcoreMesh(core_axis_name="c", subcore_axis_name="s")

  @functools.partial(
      pl.kernel,
      mesh=mesh,
      out_type=jax.ShapeDtypeStruct((B, D), jnp.float32),
      scratch_types=[
          pltpu.VMEM((b_per_w,), jnp.int32),
      ]
      + [pltpu.VMEM((K, D), jnp.float32) for _ in range(NBUF)]
      + [pltpu.SemaphoreType.DMA for _ in range(2 * NBUF)],
  )
  def k(idx_hbm, table_hbm, out_hbm, idx_v, *bufs_and_sems):
    bufs = bufs_and_sems[:NBUF]
    gsem = bufs_and_sems[NBUF : 2 * NBUF]
    wsem = bufs_and_sems[2 * NBUF :]
    wid = lax.axis_index("s") * NC + lax.axis_index("c")
    base = wid * b_per_w
    pltpu.sync_copy(idx_hbm.at[pl.ds(base, b_per_w)], idx_v)

    def gather(off, b):
      pltpu.async_copy(table_hbm.at[idx_v.at[pl.ds(off, K)]], bufs[b], gsem[b])

    def gwait(b):
      pltpu.make_async_copy(table_hbm.at[pl.ds(0, K)], bufs[b], gsem[b]).wait()

    def wstart(off, b):
      pltpu.async_copy(bufs[b], out_hbm.at[pl.ds(base + off, K)], wsem[b])

    def wwait(b):
      pltpu.make_async_copy(
          table_hbm.at[pl.ds(0, K)], bufs[b], wsem[b]
      ).wait()

    def gstep(ch, b):
      gwait(b)
      wstart(ch * K, b)
      wwait((b + 2) % 3)
      gather((ch + 2) * K, (b + 2) % 3)

    gather(0, 0)
    gather(K, 1)
    gwait(0)
    wstart(0, 0)
    gather(2 * K, 2)

    m = (n_chunks - 3) // 3

    def body(i, carry):
      ch = 1 + 3 * i
      gstep(ch, 1)
      gstep(ch + 1, 2)
      gstep(ch + 2, 0)
      return carry

    lax.fori_loop(0, m, body, 0)
    for ch in range(1 + 3 * m, n_chunks - 2):
      gstep(ch, ch % 3)
    for ch in (n_chunks - 2, n_chunks - 1):
      gwait(ch % 3)
      wstart(ch * K, ch % 3)
    for ch in (n_chunks - 3, n_chunks - 2, n_chunks - 1):
      wwait(ch % 3)

  return k


def kernel(x, table):
  V, D = table.shape
  B = x.size
  flat_idx = x.reshape((B,)).astype(jnp.int32)
  out = _make_gather(V, D, B)(flat_idx, table)
  return out.reshape(x.shape + (D,))


# speedup vs baseline: 1.0146x; 1.0146x over previous
import functools

import jax
import jax.numpy as jnp
from jax import lax
from jax.experimental import pallas as pl
from jax.experimental.pallas import tpu as pltpu
from jax.experimental.pallas import tpu_sc as plsc


@functools.lru_cache(maxsize=None)
def _make_gather(V, D, B):
  info = plsc.get_sparse_core_info()
  NC, NS = info.num_cores, info.num_subcores
  NW = NC * NS
  assert B % NW == 0
  b_per_w = B // NW
  K = 16
  NBUF = 3
  n_chunks = b_per_w // K
  assert b_per_w % K == 0 and n_chunks >= 2 * NBUF
  mesh = plsc.VectorSubcoreMesh(core_axis_name="c", subcore_axis_name="s")

  @functools.partial(
      pl.kernel,
      mesh=mesh,
      out_type=jax.ShapeDtypeStruct((B, D), jnp.float32),
      scratch_types=[
          pltpu.VMEM((b_per_w,), jnp.int32),
      ]
      + [pltpu.VMEM((K, D), jnp.float32) for _ in range(NBUF)]
      + [pltpu.SemaphoreType.DMA for _ in range(2 * NBUF)],
  )
  def k(idx_hbm, table_hbm, out_hbm, idx_v, *bufs_and_sems):
    bufs = bufs_and_sems[:NBUF]
    gsem = bufs_and_sems[NBUF : 2 * NBUF]
    wsem = bufs_and_sems[2 * NBUF :]
    wid = lax.axis_index("s") * NC + lax.axis_index("c")
    base = wid * b_per_w
    pltpu.sync_copy(idx_hbm.at[pl.ds(base, b_per_w)], idx_v)

    def gather(off, b):
      pltpu.async_copy(table_hbm.at[idx_v.at[pl.ds(off, K)]], bufs[b], gsem[b])

    def gwait(b):
      pltpu.make_async_copy(table_hbm.at[pl.ds(0, K)], bufs[b], gsem[b]).wait()

    def wstart(off, b):
      pltpu.async_copy(bufs[b], out_hbm.at[pl.ds(base + off, K)], wsem[b])

    def wwait(b):
      pltpu.make_async_copy(
          table_hbm.at[pl.ds(0, K)], bufs[b], wsem[b]
      ).wait()

    def gstep(ch, b):
      wwait((b + 2) % 3)
      gather((ch + 2) * K, (b + 2) % 3)
      gwait(b)
      wstart(ch * K, b)

    gather(0, 0)
    gather(K, 1)
    gwait(0)
    wstart(0, 0)
    gather(2 * K, 2)

    m = (n_chunks - 3) // 3

    def body(i, carry):
      ch = 1 + 3 * i
      gstep(ch, 1)
      gstep(ch + 1, 2)
      gstep(ch + 2, 0)
      return carry

    lax.fori_loop(0, m, body, 0)
    for ch in range(1 + 3 * m, n_chunks - 2):
      gstep(ch, ch % 3)
    for ch in (n_chunks - 2, n_chunks - 1):
      gwait(ch % 3)
      wstart(ch * K, ch % 3)
    for ch in (n_chunks - 3, n_chunks - 2, n_chunks - 1):
      wwait(ch % 3)

  return k


def kernel(x, table):
  V, D = table.shape
  B = x.size
  flat_idx = x.reshape((B,)).astype(jnp.int32)
  out = _make_gather(V, D, B)(flat_idx, table)
  return out.reshape(x.shape + (D,))
